# baseline (speedup 1.0000x reference)
import functools

import jax
import jax.numpy as jnp
from jax import lax
from jax.experimental import pallas as pl
from jax.experimental.pallas import tpu as pltpu
from jax.experimental.pallas import tpu_sc as plsc

B = 16384
F = 26
V = 100000
D = 16

NC = 2
NS = 16
NW = NC * NS

PAIRS = F * D
PAIRS_W = PAIRS // NW
IB = 4096
NCH = B // IB
UNROLL = 16


def _emb_kernel(xt_hbm, tt_hbm, ot_hbm, rowbuf, idxbuf, valbuf0, valbuf1, sem_r, sem_o):
    valbufs = (valbuf0, valbuf1)
    wid = lax.axis_index("s") * NC + lax.axis_index("c")
    p0 = wid * PAIRS_W
    pending = []

    def wait_slot(slot):
        for i, (s, src, dst, sem) in enumerate(pending):
            if s == slot:
                pltpu.make_async_copy(src, dst, sem).wait()
                pending.pop(i)
                return

    g = 0
    for k in range(PAIRS_W):
        p = p0 + k
        f = lax.div(p, D)
        d = lax.rem(p, D)
        if k == 0:
            pltpu.sync_copy(xt_hbm.at[f, pl.ds(0, B)], idxbuf)
        else:
            fprev = lax.div(p - 1, D)

            @pl.when(f != fprev)
            def _():
                pltpu.sync_copy(xt_hbm.at[f, pl.ds(0, B)], idxbuf)

        pltpu.sync_copy(tt_hbm.at[f, d, :], rowbuf)

        for c in range(NCH):
            slot = g % 2
            wait_slot(slot)
            vslot = valbufs[slot]

            def gbody(jj, _c=c, _vs=vslot):
                o = jj * 16
                v = idxbuf[pl.ds(_c * IB + o, 16)]
                _vs[pl.ds(o, 16)] = plsc.load_gather(rowbuf, [v])

            plsc.parallel_loop(0, IB // 16, 1, unroll=UNROLL)(gbody)
            dst = ot_hbm.at[f, d, pl.ds(c * IB, IB)]
            sem = sem_o.at[slot]
            pltpu.async_copy(vslot, dst, sem)
            pending.append((slot, vslot, dst, sem))
            g += 1

    for slot in (0, 1):
        wait_slot(slot)


@jax.jit
def kernel(X, tables):
    xt = X.T
    tt = jnp.transpose(tables, (0, 2, 1))
    mesh = plsc.VectorSubcoreMesh(core_axis_name="c", subcore_axis_name="s")
    ot = pl.kernel(
        _emb_kernel,
        out_type=jax.ShapeDtypeStruct((F, D, B), jnp.float32),
        mesh=mesh,
        scratch_types=[
            pltpu.VMEM((V,), jnp.float32),
            pltpu.VMEM((B,), jnp.int32),
            pltpu.VMEM((IB,), jnp.float32),
            pltpu.VMEM((IB,), jnp.float32),
            pltpu.SemaphoreType.DMA,
            pltpu.SemaphoreType.DMA((2,)),
        ],
        compiler_params=pltpu.CompilerParams(needs_layout_passes=False),
    )(xt, tt)
    return jnp.transpose(ot, (2, 0, 1))
